# baseline (speedup 1.0000x reference)
import functools

import jax
import jax.numpy as jnp
from jax import lax
from jax.experimental import pallas as pl
from jax.experimental.pallas import tpu as pltpu
from jax.experimental.pallas import tpu_sc as plsc

_LANES = 16
_CHUNK = 64


@functools.partial(jax.jit, static_argnums=(2, 3, 4))
def _sc_expand(input_len, table2, B, T, D):
    NC = 2
    NS = 16
    NW = NC * NS
    R = (B * T) // NW
    C = _CHUNK
    n_chunks = R // C
    w_per_b = NW // B

    mesh = plsc.VectorSubcoreMesh(core_axis_name="c", subcore_axis_name="s")

    @functools.partial(
        pl.kernel,
        mesh=mesh,
        out_type=jax.ShapeDtypeStruct((B * T, D), jnp.float32),
        scratch_types=[
            pltpu.VMEM((_LANES,), jnp.int32),
            pltpu.VMEM((C, D), jnp.float32),
            pltpu.SemaphoreType.DMA,
            pltpu.SemaphoreType.DMA,
        ],
    )
    def _k(len_hbm, tab_hbm, out_hbm, lens_v, buf, semC, semZ):
        c = lax.axis_index("c")
        s = lax.axis_index("s")
        wid = s * NC + c
        b = wid // w_per_b
        base_t = (wid % w_per_b) * R
        o_base = b * T + base_t

        pltpu.sync_copy(len_hbm, lens_v.at[pl.ds(0, B)])
        lens16 = lens_v[...]
        len_b = lens16[0]
        for bb in range(1, B):
            len_b = jnp.where(b == bb, lens16[bb], len_b)

        v = jnp.clip(len_b - base_t, 0, R)
        nfull = v // C
        m_rem = v - nfull * C
        pad0 = nfull + jnp.where(m_rem > 0, 1, 0)

        def fire_full(j, carry):
            t0 = base_t + j * C
            pltpu.make_async_copy(
                tab_hbm.at[pl.ds(t0, C)],
                out_hbm.at[pl.ds(b * T + t0, C)], semC).start()
            return carry

        lax.fori_loop(0, nfull, fire_full, 0)

        zero16 = jnp.zeros((_LANES,), jnp.float32)

        @pl.when(m_rem > 0)
        def _boundary():
            t0 = base_t + nfull * C
            pltpu.sync_copy(tab_hbm.at[pl.ds(t0, C)], buf)

            def zrow(rp, carry):
                for g in range(D // _LANES):
                    buf[rp, pl.ds(g * _LANES, _LANES)] = zero16
                return carry

            lax.fori_loop(m_rem, C, zrow, 0)
            pltpu.sync_copy(buf, out_hbm.at[pl.ds(b * T + t0, C)])

        @pl.when(pad0 < n_chunks)
        def _pads():
            def zrow(rp, carry):
                for g in range(D // _LANES):
                    buf[rp, pl.ds(g * _LANES, _LANES)] = zero16
                return carry

            lax.fori_loop(0, jnp.where(m_rem > 0, m_rem, C), zrow, 0)

            def fire_pad(j, carry):
                t0 = base_t + j * C
                pltpu.make_async_copy(
                    buf, out_hbm.at[pl.ds(b * T + t0, C)], semZ).start()
                return carry

            lax.fori_loop(pad0, n_chunks, fire_pad, 0)

            def drain_pad(j, carry):
                pltpu.make_async_copy(
                    buf, out_hbm.at[pl.ds(o_base, C)], semZ).wait()
                return carry

            lax.fori_loop(pad0, n_chunks, drain_pad, 0)

        def drain_full(j, carry):
            pltpu.make_async_copy(
                tab_hbm.at[pl.ds(base_t, C)],
                out_hbm.at[pl.ds(o_base, C)], semC).wait()
            return carry

        lax.fori_loop(0, nfull, drain_full, 0)

    return _k(input_len, table2)


def kernel(input_len, max_len, pos_table):
    del max_len
    V, D = pos_table.shape
    T = V - 1
    B = input_len.shape[0]
    table2 = pos_table[1:]
    out = _sc_expand(input_len, table2, B, T, D)
    return out.reshape(B, T, D)


# speedup vs baseline: 12.3921x; 12.3921x over previous
import functools

import jax
import jax.numpy as jnp
from jax import lax
from jax.experimental import pallas as pl
from jax.experimental.pallas import tpu as pltpu
from jax.experimental.pallas import tpu_sc as plsc

_LANES = 16
_CHUNK = 64


@functools.partial(jax.jit, static_argnums=(2, 3, 4))
def _sc_expand(input_len, table2, B, T, D):
    NC = 2
    NS = 16
    NW = NC * NS
    R = (B * T) // NW
    C = _CHUNK
    n_chunks = R // C
    w_per_b = NW // B

    mesh = plsc.VectorSubcoreMesh(core_axis_name="c", subcore_axis_name="s")

    @functools.partial(
        pl.kernel,
        mesh=mesh,
        out_type=jax.ShapeDtypeStruct((B * T, D), jnp.float32),
        scratch_types=[
            pltpu.VMEM((_LANES,), jnp.int32),
            pltpu.VMEM((C, D), jnp.float32),
            pltpu.SemaphoreType.DMA,
            pltpu.SemaphoreType.DMA,
        ],
    )
    def _k(len_hbm, tab_hbm, out_hbm, lens_v, buf, semC, semZ):
        c = lax.axis_index("c")
        s = lax.axis_index("s")
        wid = s * NC + c
        b = wid // w_per_b
        base_t = (wid % w_per_b) * R
        o_base = b * T + base_t

        pltpu.sync_copy(len_hbm, lens_v.at[pl.ds(0, B)])
        lens16 = lens_v[...]
        len_b = lens16[0]
        for bb in range(1, B):
            len_b = jnp.where(b == bb, lens16[bb], len_b)

        v = jnp.clip(len_b - base_t, 0, R)
        nfull = v // C
        m_rem = v - nfull * C
        pad0 = nfull + jnp.where(m_rem > 0, 1, 0)

        def full_body(j, carry):
            t0 = base_t + j * C
            pltpu.sync_copy(tab_hbm.at[pl.ds(t0, C)], buf)
            pltpu.sync_copy(buf, out_hbm.at[pl.ds(b * T + t0, C)])
            return carry

        lax.fori_loop(0, nfull, full_body, 0)

        zero16 = jnp.zeros((_LANES,), jnp.float32)

        @pl.when(m_rem > 0)
        def _boundary():
            t0 = base_t + nfull * C
            pltpu.sync_copy(tab_hbm.at[pl.ds(t0, C)], buf)

            def zrow(rp, carry):
                for g in range(D // _LANES):
                    buf[rp, pl.ds(g * _LANES, _LANES)] = zero16
                return carry

            lax.fori_loop(m_rem, C, zrow, 0)
            pltpu.sync_copy(buf, out_hbm.at[pl.ds(b * T + t0, C)])

        @pl.when(pad0 < n_chunks)
        def _pads():
            def zrow(rp, carry):
                for g in range(D // _LANES):
                    buf[rp, pl.ds(g * _LANES, _LANES)] = zero16
                return carry

            lax.fori_loop(0, jnp.where(m_rem > 0, m_rem, C), zrow, 0)

            def fire_pad(j, carry):
                t0 = base_t + j * C
                pltpu.make_async_copy(
                    buf, out_hbm.at[pl.ds(b * T + t0, C)], semZ).start()
                return carry

            lax.fori_loop(pad0, n_chunks, fire_pad, 0)

            def drain_pad(j, carry):
                pltpu.make_async_copy(
                    buf, out_hbm.at[pl.ds(o_base, C)], semZ).wait()
                return carry

            lax.fori_loop(pad0, n_chunks, drain_pad, 0)

    return _k(input_len, table2)


def kernel(input_len, max_len, pos_table):
    del max_len
    V, D = pos_table.shape
    T = V - 1
    B = input_len.shape[0]
    table2 = pos_table[1:]
    out = _sc_expand(input_len, table2, B, T, D)
    return out.reshape(B, T, D)
